# baseline (speedup 1.0000x reference)
import functools

import jax
import jax.numpy as jnp
from jax import lax
from jax.experimental import pallas as pl
from jax.experimental.pallas import tpu as pltpu
from jax.experimental.pallas import tpu_sc as plsc

_L = 16


def _one_lookup(idx, tab_t):
    B = idx.shape[0]
    D, V = tab_t.shape
    n_vec = B // _L
    mesh = plsc.VectorSubcoreMesh(core_axis_name="c", subcore_axis_name="s")
    info = plsc.get_sparse_core_info()

    @functools.partial(
        pl.kernel,
        mesh=mesh,
        compiler_params=pltpu.CompilerParams(
            use_tc_tiling_on_sc=False, needs_layout_passes=False),
        out_type=jax.ShapeDtypeStruct((D, B), jnp.int32),
        scratch_types=[
            pltpu.VMEM((B,), jnp.int32),
            pltpu.VMEM((V,), jnp.float32),
            pltpu.SemaphoreType.DMA,
            pltpu.SemaphoreType.DMA,
        ],
    )
    def body(idx_hbm, tab_hbm, out_hbm, iob_v, col_v, sem_i, sem_c):
        wid = lax.axis_index("s") * info.num_cores + lax.axis_index("c")
        ci = pltpu.async_copy(idx_hbm, iob_v, sem_i)
        cc = pltpu.async_copy(tab_hbm.at[wid], col_v, sem_c)
        ci.wait()
        cc.wait()

        @plsc.parallel_loop(0, n_vec, unroll=8)
        def gather_chunk(i):
            sl = pl.ds(i * _L, _L)
            iob_v[sl] = plsc.bitcast(
                plsc.load_gather(col_v, [iob_v[sl]]), jnp.int32)

        pltpu.sync_copy(iob_v, out_hbm.at[wid])

    return body(idx, tab_t)


_CHUNK = 128


def _row_lookup(idx, tab):
    B = idx.shape[0]
    V, D = tab.shape
    info = plsc.get_sparse_core_info()
    nw = info.num_cores * info.num_subcores
    b_per_w = B // nw
    n_chunks = b_per_w // _CHUNK
    mesh = plsc.VectorSubcoreMesh(core_axis_name="c", subcore_axis_name="s")

    @functools.partial(
        pl.kernel,
        mesh=mesh,
        compiler_params=pltpu.CompilerParams(use_tc_tiling_on_sc=False),
        out_type=jax.ShapeDtypeStruct((B, D), jnp.float32),
        scratch_types=[
            pltpu.VMEM((b_per_w,), jnp.int32),
            pltpu.VMEM((b_per_w, D), jnp.float32),
            pltpu.SemaphoreType.DMA,
        ],
    )
    def body(idx_hbm, tab_hbm, out_hbm, idx_v, rows_v, sem):
        wid = lax.axis_index("s") * info.num_cores + lax.axis_index("c")
        base = wid * b_per_w
        pltpu.sync_copy(idx_hbm.at[pl.ds(base, b_per_w)], idx_v)
        copies = []
        for j in range(n_chunks):
            sl = pl.ds(j * _CHUNK, _CHUNK)
            copies.append(
                pltpu.async_copy(tab_hbm.at[idx_v.at[sl]], rows_v.at[sl],
                                 sem))
        for c in copies:
            c.wait()
        pltpu.sync_copy(rows_v, out_hbm.at[pl.ds(base, b_per_w)])

    return body(idx, tab)


@jax.jit
def _lookup(hair, eyes, hair_table, eye_table):
    ht = _one_lookup(hair, hair_table.T)
    et = _row_lookup(eyes, eye_table)
    ht = lax.bitcast_convert_type(ht, jnp.float32)
    return ht.T, et


def kernel(hair, eyes, hair_table, eye_table):
    return _lookup(hair, eyes, hair_table, eye_table)


# speedup vs baseline: 1.2820x; 1.2820x over previous
import functools

import jax
import jax.numpy as jnp
from jax import lax
from jax.experimental import pallas as pl
from jax.experimental.pallas import tpu as pltpu
from jax.experimental.pallas import tpu_sc as plsc

_L = 16


def _one_lookup(idx, tab_t):
    B = idx.shape[0]
    D, V = tab_t.shape
    n_vec = B // _L
    mesh = plsc.VectorSubcoreMesh(core_axis_name="c", subcore_axis_name="s")
    info = plsc.get_sparse_core_info()

    @functools.partial(
        pl.kernel,
        mesh=mesh,
        compiler_params=pltpu.CompilerParams(
            use_tc_tiling_on_sc=False, needs_layout_passes=False),
        out_type=jax.ShapeDtypeStruct((D, B), jnp.int32),
        scratch_types=[
            pltpu.VMEM((B,), jnp.int32),
            pltpu.VMEM((V,), jnp.float32),
            pltpu.SemaphoreType.DMA,
            pltpu.SemaphoreType.DMA,
        ],
    )
    def body(idx_hbm, tab_hbm, out_hbm, iob_v, col_v, sem_i, sem_c):
        wid = lax.axis_index("s") * info.num_cores + lax.axis_index("c")
        ci = pltpu.async_copy(idx_hbm, iob_v, sem_i)
        cc = pltpu.async_copy(tab_hbm.at[wid], col_v, sem_c)
        ci.wait()
        cc.wait()

        @plsc.parallel_loop(0, n_vec, unroll=16)
        def gather_chunk(i):
            sl = pl.ds(i * _L, _L)
            iob_v[sl] = plsc.bitcast(
                plsc.load_gather(col_v, [iob_v[sl]]), jnp.int32)

        pltpu.sync_copy(iob_v, out_hbm.at[wid])

    return body(idx, tab_t)


@jax.jit
def _lookup(hair, eyes, hair_table, eye_table):
    ht = _one_lookup(hair, hair_table.T)
    et = _one_lookup(eyes, eye_table.T)
    ht = lax.bitcast_convert_type(ht, jnp.float32)
    et = lax.bitcast_convert_type(et, jnp.float32)
    return ht.T, et.T


def kernel(hair, eyes, hair_table, eye_table):
    return _lookup(hair, eyes, hair_table, eye_table)
